# Initial kernel scaffold; baseline (speedup 1.0000x reference)
import jax
import jax.numpy as jnp
from jax.experimental import pallas as pl


def kernel(x, edge_index, W1, b1, W2, b2):
    raise NotImplementedError("write your pallas kernel here")


# speedup vs baseline: 12.6837x; 12.6837x over previous
import functools

import jax
import jax.numpy as jnp
from jax import lax
from jax.experimental import pallas as pl
from jax.experimental.pallas import tpu as pltpu
from jax.experimental.pallas import tpu_sc as plsc

N = 10000
E = 320000
F = 128
H = 256

NC, NS = 2, 16
CHUNK = 128
T_CH = -(-E // (CHUNK * NC * NS))
C_TOT = T_CH * NC * NS
E_PAD = C_TOT * CHUNK
N_PAD = 10240
SLICE = N_PAD // NS
ROWS = 2000

_mesh = plsc.VectorSubcoreMesh(
    core_axis_name="c", subcore_axis_name="s", num_cores=NC, num_subcores=NS)


@functools.partial(
    pl.kernel,
    out_type=jax.ShapeDtypeStruct((NC, NS, SLICE), jnp.float32),
    mesh=_mesh,
    scratch_types=[
        pltpu.VMEM((1, CHUNK), jnp.int32),
        pltpu.VMEM((1, CHUNK), jnp.float32),
        pltpu.VMEM_SHARED((N_PAD,), jnp.float32),
    ],
)
def _sc_degree(dst_hbm, zeros_hbm, out_hbm, idx_v, ones_v, deg_sh):
    c = lax.axis_index("c")
    s = lax.axis_index("s")
    pltpu.sync_copy(zeros_hbm.at[pl.ds(s * SLICE, SLICE)],
                    deg_sh.at[pl.ds(s * SLICE, SLICE)])
    for j in range(CHUNK // 16):
        ones_v[0, pl.ds(j * 16, 16)] = jnp.full((16,), 1.0, jnp.float32)
    plsc.subcore_barrier()
    base = c * (NS * T_CH) + s * T_CH

    def body(j, carry):
        pltpu.sync_copy(dst_hbm.at[base + j], idx_v.at[0])
        pltpu.sync_copy(ones_v.at[0], deg_sh.at[idx_v.at[0]], add=True)
        return carry

    lax.fori_loop(0, T_CH, body, 0)
    plsc.subcore_barrier()
    pltpu.sync_copy(deg_sh.at[pl.ds(s * SLICE, SLICE)], out_hbm.at[c, s])


@functools.partial(
    pl.kernel,
    out_type=jax.ShapeDtypeStruct((NC, NS, SLICE, F), jnp.float32),
    mesh=_mesh,
    scratch_types=[
        pltpu.VMEM((1, CHUNK), jnp.int32),
        pltpu.VMEM((1, CHUNK), jnp.int32),
        pltpu.VMEM((CHUNK, F), jnp.float32),
        pltpu.VMEM_SHARED((N_PAD, F), jnp.float32),
        pltpu.SemaphoreType.DMA,
    ],
)
def _sc_aggregate(src_hbm, dst_hbm, z_hbm, zeros_hbm, out_hbm,
                  si_v, di_v, msg_v, acc_sh, sem):
    c = lax.axis_index("c")
    s = lax.axis_index("s")
    pltpu.sync_copy(zeros_hbm.at[pl.ds(s * SLICE, SLICE)],
                    acc_sh.at[pl.ds(s * SLICE, SLICE)])
    plsc.subcore_barrier()
    base = c * (NS * T_CH) + s * T_CH

    def body(j, carry):
        pltpu.sync_copy(src_hbm.at[base + j], si_v.at[0])
        pltpu.sync_copy(dst_hbm.at[base + j], di_v.at[0])
        pltpu.async_copy(z_hbm.at[si_v.at[0]], msg_v, sem).wait()
        pltpu.sync_copy(msg_v, acc_sh.at[di_v.at[0]], add=True)
        return carry

    lax.fori_loop(0, T_CH, body, 0)
    plsc.subcore_barrier()
    pltpu.sync_copy(acc_sh.at[pl.ds(s * SLICE, SLICE)], out_hbm.at[c, s])


def _tc_pre_body(d0, d1, x, dis_ref, y0_ref):
    dis = lax.rsqrt(d0[...] + d1[...] + 1.0)
    dis_ref[...] = dis
    y0_ref[...] = x[...] * dis


def _tc_mid_body(a0, a1, y0, dis, w1, b1, w2, y1_ref):
    out1 = (a0[...] + a1[...] + y0[...]) * dis[...]
    h = jnp.dot(out1, w1[...], preferred_element_type=jnp.float32) + b1[...]
    h = jnp.maximum(h, 0.0)
    y1_ref[...] = jnp.dot(h, w2[...], preferred_element_type=jnp.float32) * dis[...]


def _tc_fin_body(a0, a1, y1, dis, b2, out_ref):
    out_ref[...] = (a0[...] + a1[...] + y1[...]) * dis[...] + b2[...]


def _row_spec(cols):
    return pl.BlockSpec((ROWS, cols), lambda i: (i, 0))


def _full_spec(r, c):
    return pl.BlockSpec((r, c), lambda i: (0, 0))


_GRID = (N // ROWS,)

_tc_pre = pl.pallas_call(
    _tc_pre_body,
    grid=_GRID,
    in_specs=[_row_spec(1), _row_spec(1), _row_spec(F)],
    out_specs=[_row_spec(1), _row_spec(F)],
    out_shape=[jax.ShapeDtypeStruct((N, 1), jnp.float32),
               jax.ShapeDtypeStruct((N, F), jnp.float32)],
)

_tc_mid = pl.pallas_call(
    _tc_mid_body,
    grid=_GRID,
    in_specs=[_row_spec(F), _row_spec(F), _row_spec(F), _row_spec(1),
              _full_spec(F, H), _full_spec(1, H), _full_spec(H, F)],
    out_specs=_row_spec(F),
    out_shape=jax.ShapeDtypeStruct((N, F), jnp.float32),
)

_tc_fin = pl.pallas_call(
    _tc_fin_body,
    grid=_GRID,
    in_specs=[_row_spec(F), _row_spec(F), _row_spec(F), _row_spec(1),
              _full_spec(1, F)],
    out_specs=_row_spec(F),
    out_shape=jax.ShapeDtypeStruct((N, F), jnp.float32),
)


def kernel(x, edge_index, W1, b1, W2, b2):
    ei = edge_index.astype(jnp.int32)
    n_dummy = E_PAD - E
    pad_dst = N + (jnp.arange(n_dummy, dtype=jnp.int32) % (N_PAD - N))
    pad_src = jnp.full((n_dummy,), N, jnp.int32)
    src = jnp.concatenate([ei[0], pad_src]).reshape(C_TOT, CHUNK)
    dst = jnp.concatenate([ei[1], pad_dst]).reshape(C_TOT, CHUNK)

    zeros1 = jnp.zeros((N_PAD,), jnp.float32)
    zeros2 = jnp.zeros((N_PAD, F), jnp.float32)

    deg_parts = _sc_degree(dst, zeros1).reshape(NC, N_PAD)
    d0 = deg_parts[0, :N].reshape(N, 1)
    d1 = deg_parts[1, :N].reshape(N, 1)

    dis, y0 = _tc_pre(d0, d1, x)

    y0_pad = jnp.zeros((N_PAD, F), jnp.float32).at[:N].set(y0)
    acc = _sc_aggregate(src, dst, y0_pad, zeros2).reshape(NC, N_PAD, F)

    y1 = _tc_mid(acc[0, :N], acc[1, :N], y0, dis,
                 W1, b1.reshape(1, H), W2)

    y1_pad = jnp.zeros((N_PAD, F), jnp.float32).at[:N].set(y1)
    acc2 = _sc_aggregate(src, dst, y1_pad, zeros2).reshape(NC, N_PAD, F)

    out = _tc_fin(acc2[0, :N], acc2[1, :N], y1, dis, b2.reshape(1, F))
    return out
